# Initial kernel scaffold; baseline (speedup 1.0000x reference)
import jax
import jax.numpy as jnp
from jax.experimental import pallas as pl


def kernel(x, shared_gate_w, shared_up_w, shared_down_w, gate_w, logit_bias, null_logit, W_gate, W_up, W_down):
    raise NotImplementedError("write your pallas kernel here")


# speedup vs baseline: 1.6338x; 1.6338x over previous
import functools

import jax
import jax.numpy as jnp
from jax import lax
from jax.experimental import pallas as pl
from jax.experimental.pallas import tpu as pltpu

_T = 2048
_D = 1024
_DH = 512
_DS = 2048
_E = 8
_NULL_COPIES = 8
_RHO = 0.5


def _silu(v):
    return v * jax.nn.sigmoid(v)


def _gating_body(x_ref, gw_ref, bias_ref, null_ref, w_ref, aux_ref):
    x = x_ref[...]
    logits = jnp.dot(x, gw_ref[...], preferred_element_type=jnp.float32)
    logits = logits + bias_ref[...]
    n = null_ref[0, 0]
    T, E = logits.shape

    iota = lax.broadcasted_iota(jnp.int32, (T, E), 1)
    v1 = jnp.max(logits, axis=1, keepdims=True)
    e1 = jnp.min(jnp.where(logits == v1, iota, E), axis=1, keepdims=True)
    l2 = jnp.where(iota == e1, -jnp.inf, logits)
    v2 = jnp.max(l2, axis=1, keepdims=True)
    e2 = jnp.min(jnp.where(l2 == v2, iota, E), axis=1, keepdims=True)

    t1_real = v1 >= n
    t2_real = jnp.logical_and(t1_real, v2 >= n)

    w1_both = 1.0 / (1.0 + jnp.exp(v2 - v1))
    w1 = jnp.where(t1_real, jnp.where(t2_real, w1_both, 1.0), 0.0)
    w2 = jnp.where(t2_real, 1.0 - w1_both, 0.0)
    w_full = (jnp.where(iota == e1, w1, 0.0) + jnp.where(iota == e2, w2, 0.0))
    w_ref[...] = w_full

    p = jnp.exp(logits - v1)
    probs_real = p / jnp.sum(p, axis=1, keepdims=True)
    P_real = jnp.sum(probs_real, axis=0) / T

    sel1 = jnp.where(jnp.logical_and(iota == e1, t1_real), 1.0, 0.0)
    sel2 = jnp.where(jnp.logical_and(iota == e2, t2_real), 1.0, 0.0)
    counts = jnp.sum(sel1 + sel2, axis=0)
    total_real = jnp.maximum(jnp.sum(counts), 1e-6)
    f_real = counts / total_real
    L_bal = E * jnp.sum(f_real * P_real)

    null_slots = jnp.sum(jnp.where(t1_real, 0.0, 1.0) + jnp.where(t2_real, 0.0, 1.0))
    null_rate = null_slots / (T * 2)
    L_null = (null_rate - _RHO) ** 2

    m = jnp.maximum(v1, n)
    s_all = (jnp.sum(jnp.exp(logits - m), axis=1, keepdims=True)
             + _NULL_COPIES * jnp.exp(n - m))
    lse = m + jnp.log(s_all)
    L_z = jnp.sum(lse * lse) / T

    aux_ref[0, 0] = 0.02 * L_bal + 0.001 * L_z + 0.01 * L_null


def _gating(x2d, gate_w, logit_bias, null_logit):
    return pl.pallas_call(
        _gating_body,
        out_shape=(
            jax.ShapeDtypeStruct((_T, _E), jnp.float32),
            jax.ShapeDtypeStruct((1, 1), jnp.float32),
        ),
        in_specs=[
            pl.BlockSpec(memory_space=pltpu.VMEM),
            pl.BlockSpec(memory_space=pltpu.VMEM),
            pl.BlockSpec(memory_space=pltpu.VMEM),
            pl.BlockSpec(memory_space=pltpu.VMEM),
        ],
        out_specs=(
            pl.BlockSpec(memory_space=pltpu.VMEM),
            pl.BlockSpec(memory_space=pltpu.SMEM),
        ),
    )(x2d, gate_w, logit_bias.reshape(1, _E), null_logit.reshape(1, 1))


def _shared_body(x_ref, gw_ref, uw_ref, dw_ref, out_ref):
    x = x_ref[...]
    g = jnp.dot(x, gw_ref[...], preferred_element_type=jnp.float32)
    u = jnp.dot(x, uw_ref[...], preferred_element_type=jnp.float32)
    h = _silu(g) * u
    out_ref[...] = jnp.dot(h, dw_ref[...], preferred_element_type=jnp.float32)


def _shared_ffn(x2d, gw, uw, dw, tb=512):
    grid = (_T // tb,)
    return pl.pallas_call(
        _shared_body,
        grid=grid,
        out_shape=jax.ShapeDtypeStruct((_T, _D), jnp.float32),
        in_specs=[
            pl.BlockSpec((tb, _D), lambda t: (t, 0)),
            pl.BlockSpec((_D, _DS), lambda t: (0, 0)),
            pl.BlockSpec((_D, _DS), lambda t: (0, 0)),
            pl.BlockSpec((_DS, _D), lambda t: (0, 0)),
        ],
        out_specs=pl.BlockSpec((tb, _D), lambda t: (t, 0)),
    )(x2d, gw, uw, dw)


def _experts_body(x_ref, wg_ref, wu_ref, wd_ref, wfull_ref, shared_ref,
                  out_ref, *, tb):
    e = pl.program_id(0)
    t = pl.program_id(1)
    xb = x_ref[pl.ds(t * tb, tb), :]
    g = jnp.dot(xb, wg_ref[0], preferred_element_type=jnp.float32)
    u = jnp.dot(xb, wu_ref[0], preferred_element_type=jnp.float32)
    h = _silu(g) * u
    wblk = wfull_ref[pl.ds(t * tb, tb), :]
    eiota = lax.broadcasted_iota(jnp.int32, wblk.shape, 1)
    w = jnp.sum(jnp.where(eiota == e, wblk, 0.0), axis=1, keepdims=True)
    h = h * w
    contrib = jnp.dot(h, wd_ref[0], preferred_element_type=jnp.float32)

    @pl.when(e == 0)
    def _init():
        out_ref[pl.ds(t * tb, tb), :] = shared_ref[pl.ds(t * tb, tb), :] + contrib

    @pl.when(e > 0)
    def _acc():
        cur = out_ref[pl.ds(t * tb, tb), :]
        out_ref[pl.ds(t * tb, tb), :] = cur + contrib


def _experts(x2d, W_gate, W_up, W_down, wfull, shared_out, tb=256):
    grid = (_E, _T // tb)
    body = functools.partial(_experts_body, tb=tb)
    return pl.pallas_call(
        body,
        grid=grid,
        out_shape=jax.ShapeDtypeStruct((_T, _D), jnp.float32),
        in_specs=[
            pl.BlockSpec((_T, _D), lambda e, t: (0, 0)),
            pl.BlockSpec((1, _D, _DH), lambda e, t: (e, 0, 0)),
            pl.BlockSpec((1, _D, _DH), lambda e, t: (e, 0, 0)),
            pl.BlockSpec((1, _DH, _D), lambda e, t: (e, 0, 0)),
            pl.BlockSpec((_T, _E), lambda e, t: (0, 0)),
            pl.BlockSpec((_T, _D), lambda e, t: (0, 0)),
        ],
        out_specs=pl.BlockSpec((_T, _D), lambda e, t: (0, 0)),
    )(x2d, W_gate, W_up, W_down, wfull, shared_out)


def kernel(x, shared_gate_w, shared_up_w, shared_down_w, gate_w, logit_bias,
           null_logit, W_gate, W_up, W_down):
    Bx, Tx, D = x.shape
    x2d = x.reshape(_T, _D)
    wfull, aux = _gating(x2d, gate_w, logit_bias, null_logit)
    shared_out = _shared_ffn(x2d, shared_gate_w, shared_up_w, shared_down_w)
    y = _experts(x2d, W_gate, W_up, W_down, wfull, shared_out)
    return y.reshape(Bx, Tx, D), aux[0, 0]
